# Initial kernel scaffold; baseline (speedup 1.0000x reference)
import jax
import jax.numpy as jnp
from jax.experimental import pallas as pl


def kernel(card_ids, card_features, table, W, b):
    raise NotImplementedError("write your pallas kernel here")


# speedup vs baseline: 2.7902x; 2.7902x over previous
import functools

import jax
import jax.numpy as jnp
from jax import lax
from jax.experimental import pallas as pl
from jax.experimental.pallas import tpu as pltpu
from jax.experimental.pallas import tpu_sc as plsc

NUM_CARDS = 100000
CARD_ID_DIM = 64
HIDDEN_DIM = 128
BATCH = 4096
SEQ_LEN = 200
FEAT_EXTRA = 11
TOTAL = BATCH * SEQ_LEN

NC = 2
NS = 16
NW = NC * NS
PER_W = TOTAL // NW
CHUNK = 128
NCHUNK = PER_W // CHUNK


def _gather_body(ids_hbm, table_hbm, out_hbm, idx_v, rows_v, sem_i, sem_g):
    wid = lax.axis_index("s") * NC + lax.axis_index("c")
    base = wid * PER_W

    pltpu.async_copy(ids_hbm.at[wid], idx_v, sem_i).wait()

    def body(i, _):
        pltpu.async_copy(table_hbm.at[idx_v.at[i]], rows_v, sem_g).wait()
        pltpu.sync_copy(rows_v, out_hbm.at[pl.ds(base + i * CHUNK, CHUNK)])
        return ()

    lax.fori_loop(0, NCHUNK, body, (), unroll=False)


@jax.jit
def _sc_gather(ids3, table):
    mesh = plsc.VectorSubcoreMesh(
        core_axis_name="c", subcore_axis_name="s", num_cores=NC, num_subcores=NS
    )
    return pl.kernel(
        _gather_body,
        out_type=jax.ShapeDtypeStruct((TOTAL, CARD_ID_DIM), jnp.float32),
        mesh=mesh,
        compiler_params=pltpu.CompilerParams(use_tc_tiling_on_sc=False),
        scratch_types=[
            pltpu.VMEM((NCHUNK, CHUNK), jnp.int32),
            pltpu.VMEM((CHUNK, CARD_ID_DIM), jnp.float32),
            pltpu.SemaphoreType.DMA,
            pltpu.SemaphoreType.DMA,
        ],
    )(ids3, table)


R_BLOCK = 2048


def _mm_body(g_ref, f_ref, wid_ref, wf_ref, b_ref, o_ref):
    acc = jnp.dot(g_ref[...], wid_ref[...], preferred_element_type=jnp.float32)
    acc = acc + jnp.dot(f_ref[...], wf_ref[...], preferred_element_type=jnp.float32)
    o_ref[...] = acc + b_ref[...]


@jax.jit
def _tc_project(g, feat_flat, wid_t, wf_t, b2):
    grid = (TOTAL // R_BLOCK,)
    return pl.pallas_call(
        _mm_body,
        grid=grid,
        in_specs=[
            pl.BlockSpec((R_BLOCK, CARD_ID_DIM), lambda i: (i, 0)),
            pl.BlockSpec((R_BLOCK, FEAT_EXTRA), lambda i: (i, 0)),
            pl.BlockSpec((CARD_ID_DIM, HIDDEN_DIM), lambda i: (0, 0)),
            pl.BlockSpec((FEAT_EXTRA, HIDDEN_DIM), lambda i: (0, 0)),
            pl.BlockSpec((1, HIDDEN_DIM), lambda i: (0, 0)),
        ],
        out_specs=pl.BlockSpec((R_BLOCK, HIDDEN_DIM), lambda i: (i, 0)),
        out_shape=jax.ShapeDtypeStruct((TOTAL, HIDDEN_DIM), jnp.float32),
    )(g, feat_flat, wid_t, wf_t, b2)


def kernel(card_ids, card_features, table, W, b):
    ids3 = card_ids.reshape(NW, NCHUNK, CHUNK).astype(jnp.int32)
    g = _sc_gather(ids3, table)
    feat_flat = card_features.reshape(TOTAL, FEAT_EXTRA)
    wid_t = W[:, :CARD_ID_DIM].T
    wf_t = W[:, CARD_ID_DIM:].T
    b2 = b.reshape(1, HIDDEN_DIM)
    out = _tc_project(g, feat_flat, wid_t, wf_t, b2)
    return out.reshape(BATCH, SEQ_LEN, HIDDEN_DIM)


# speedup vs baseline: 3.4346x; 1.2310x over previous
import jax
import jax.numpy as jnp
from jax import lax
from jax.experimental import pallas as pl
from jax.experimental.pallas import tpu as pltpu
from jax.experimental.pallas import tpu_sc as plsc

NUM_CARDS = 100000
CARD_ID_DIM = 64
HIDDEN_DIM = 128
BATCH = 4096
SEQ_LEN = 200
FEAT_EXTRA = 11
TOTAL = BATCH * SEQ_LEN

NC = 2
NS = 16
NW = NC * NS
PER_W = TOTAL // NW
CHUNK = 128
NCHUNK = PER_W // CHUNK

BB = 512
SB = 8
HALF = BB // 2


def _gather_body(ids_hbm, table_hbm, out_hbm, idx_v, rows_v, sem_i, sem_g):
    wid = lax.axis_index("s") * NC + lax.axis_index("c")
    base = wid * PER_W

    pltpu.async_copy(ids_hbm.at[wid], idx_v, sem_i).wait()

    def body(i, _):
        pltpu.async_copy(table_hbm.at[idx_v.at[i]], rows_v, sem_g).wait()
        pltpu.sync_copy(rows_v, out_hbm.at[pl.ds(base + i * CHUNK, CHUNK)])
        return ()

    lax.fori_loop(0, NCHUNK, body, (), unroll=False)


@jax.jit
def _sc_gather(ids3, table):
    mesh = plsc.VectorSubcoreMesh(
        core_axis_name="c", subcore_axis_name="s", num_cores=NC, num_subcores=NS
    )
    return pl.kernel(
        _gather_body,
        out_type=jax.ShapeDtypeStruct((TOTAL, CARD_ID_DIM), jnp.float32),
        mesh=mesh,
        compiler_params=pltpu.CompilerParams(use_tc_tiling_on_sc=False),
        scratch_types=[
            pltpu.VMEM((NCHUNK, CHUNK), jnp.int32),
            pltpu.VMEM((CHUNK, CARD_ID_DIM), jnp.float32),
            pltpu.SemaphoreType.DMA,
            pltpu.SemaphoreType.DMA,
        ],
    )(ids3, table)


def _mm_body(g_ref, f_ref, wid_ref, wf_ref, b_ref, o_ref):
    bvec = b_ref[...]
    wid = wid_ref[...]
    wf = wf_ref[...]
    for s in range(SB):
        g = g_ref[s]
        oe = jnp.dot(g[:, :CARD_ID_DIM], wid, preferred_element_type=jnp.float32)
        oo = jnp.dot(g[:, CARD_ID_DIM:], wid, preferred_element_type=jnp.float32)
        fts = f_ref[:, s, :]
        of = lax.dot_general(
            fts, wf,
            dimension_numbers=(((0,), (0,)), ((), ())),
            preferred_element_type=jnp.float32,
        )
        o_ref[0:HALF, s, :] = oe + of[0:HALF] + bvec
        o_ref[HALF:BB, s, :] = oo + of[HALF:BB] + bvec


@jax.jit
def _tc_project(g3, ft, wid_t, wf_t, b2):
    grid = (BATCH // BB, SEQ_LEN // SB)
    return pl.pallas_call(
        _mm_body,
        grid=grid,
        in_specs=[
            pl.BlockSpec((SB, HALF, HIDDEN_DIM), lambda ib, isq: (isq, ib, 0)),
            pl.BlockSpec((FEAT_EXTRA, SB, BB), lambda ib, isq: (0, isq, ib)),
            pl.BlockSpec((CARD_ID_DIM, HIDDEN_DIM), lambda ib, isq: (0, 0)),
            pl.BlockSpec((FEAT_EXTRA, HIDDEN_DIM), lambda ib, isq: (0, 0)),
            pl.BlockSpec((1, HIDDEN_DIM), lambda ib, isq: (0, 0)),
        ],
        out_specs=pl.BlockSpec(
            (BB, SB, HIDDEN_DIM), lambda ib, isq: (ib, isq, 0)
        ),
        out_shape=jax.ShapeDtypeStruct((BATCH, SEQ_LEN, HIDDEN_DIM), jnp.float32),
    )(g3, ft, wid_t, wf_t, b2)


def kernel(card_ids, card_features, table, W, b):
    ids_perm = (
        card_ids.T.astype(jnp.int32)
        .reshape(SEQ_LEN, BATCH // BB, 2, HALF)
        .transpose(0, 1, 3, 2)
        .reshape(NW, NCHUNK, CHUNK)
    )
    g = _sc_gather(ids_perm, table)
    g3 = g.reshape(SEQ_LEN, BATCH // 2, HIDDEN_DIM)
    ft = jnp.transpose(card_features, (2, 1, 0))
    wid_t = W[:, :CARD_ID_DIM].T
    wf_t = W[:, CARD_ID_DIM:].T
    b2 = b.reshape(1, HIDDEN_DIM)
    return _tc_project(g3, ft, wid_t, wf_t, b2)
